# baseline (speedup 1.0000x reference)
import functools

import jax
import jax.numpy as jnp
from jax import lax
from jax.experimental import pallas as pl
from jax.experimental.pallas import tpu as pltpu
from jax.experimental.pallas import tpu_sc as plsc

D = 64
H = 256
B = 16384
T = 6
NC = 2
NS = 16
NW = NC * NS
BW = B // NW

IDX_MINOR = 128
IDX_SLICES = BW * T // IDX_MINOR
CHUNK_TEAMS = 64
CHUNK_ROWS = CHUNK_TEAMS * T
SLICES_PER_CHUNK = CHUNK_ROWS // IDX_MINOR
NCHUNKS = BW // CHUNK_TEAMS

_INV_T = 1.0 / T


def _pool_body(idx_hbm, table_hbm, out_hbm,
               idx_v, rows_a, rows_b, pool_v, sem_a, sem_b):
    c = lax.axis_index("c")
    s = lax.axis_index("s")
    wid = s * NC + c
    base_team = wid * BW

    pltpu.sync_copy(idx_hbm.at[wid], idx_v)

    rows = (rows_a, rows_b)
    sems = (sem_a, sem_b)

    def fire(ci):
        buf = ci % 2
        cps = []
        for j in range(SLICES_PER_CHUNK):
            cps.append(pltpu.async_copy(
                table_hbm.at[idx_v.at[ci * SLICES_PER_CHUNK + j]],
                rows[buf].at[pl.ds(j * IDX_MINOR, IDX_MINOR)],
                sems[buf]))
        return cps

    pending = fire(0)
    for ci in range(NCHUNKS):
        buf = ci % 2
        nxt = fire(ci + 1) if ci + 1 < NCHUNKS else None
        for cp in pending:
            cp.wait()
        pending = nxt
        rv = rows[buf]

        def team(t, carry):
            rbase = t * T
            for d in range(D // 16):
                col = pl.ds(d * 16, 16)
                acc = rv[rbase, col]
                for r in range(1, T):
                    acc = acc + rv[rbase + r, col]
                pool_v[t, col] = acc * _INV_T
            return carry

        lax.fori_loop(0, CHUNK_TEAMS, team, 0, unroll=False)
        pltpu.sync_copy(
            pool_v,
            out_hbm.at[pl.ds(base_team + ci * CHUNK_TEAMS, CHUNK_TEAMS)])


@functools.cache
def _make_pool_call():
    return pl.kernel(
        _pool_kernel_fn,
        mesh=plsc.VectorSubcoreMesh(core_axis_name="c", subcore_axis_name="s"),
        out_type=jax.ShapeDtypeStruct((B, D), jnp.float32),
        scratch_types=[
            pltpu.VMEM((IDX_SLICES, IDX_MINOR), jnp.int32),
            pltpu.VMEM((CHUNK_ROWS, D), jnp.float32),
            pltpu.VMEM((CHUNK_ROWS, D), jnp.float32),
            pltpu.VMEM((CHUNK_TEAMS, D), jnp.float32),
            pltpu.SemaphoreType.DMA,
            pltpu.SemaphoreType.DMA,
        ],
        compiler_params=pltpu.CompilerParams(use_tc_tiling_on_sc=False),
    )


def _pool_kernel_fn(idx, table, out, *rest):
    _pool_body(idx, table, out, *rest)


def _gelu(x):
    return 0.5 * x * (1.0 + lax.erf(x * 0.7071067811865476))


def _mlp_body(x_ref, w1_ref, b1_ref, w2_ref, b2_ref, w3_ref, b3_ref, o_ref):
    x = x_ref[...]
    h = jnp.dot(x, w1_ref[...], preferred_element_type=jnp.float32) + b1_ref[...]
    h = _gelu(h)
    h = jnp.dot(h, w2_ref[...], preferred_element_type=jnp.float32) + b2_ref[...]
    h = _gelu(h)
    logits = jnp.sum(h * w3_ref[...], axis=1) + b3_ref[0]
    o_ref[...] = 1.0 / (1.0 + jnp.exp(-logits))


BT = 1024

_mlp_call = pl.pallas_call(
    _mlp_body,
    grid=(B // BT,),
    in_specs=[
        pl.BlockSpec((BT, D), lambda i: (i, 0)),
        pl.BlockSpec((D, H), lambda i: (0, 0)),
        pl.BlockSpec((1, H), lambda i: (0, 0)),
        pl.BlockSpec((H, H), lambda i: (0, 0)),
        pl.BlockSpec((1, H), lambda i: (0, 0)),
        pl.BlockSpec((1, H), lambda i: (0, 0)),
        pl.BlockSpec(memory_space=pltpu.SMEM),
    ],
    out_specs=pl.BlockSpec((BT,), lambda i: (i,)),
    out_shape=jax.ShapeDtypeStruct((B,), jnp.float32),
)


def kernel(set_indices, emb_table, W1, b1, W2, b2, W3, b3):
    idx = set_indices.astype(jnp.int32).reshape(NW, IDX_SLICES, IDX_MINOR)
    team_emb = _make_pool_call()(idx, emb_table)
    return _mlp_call(team_emb, W1, b1.reshape(1, H), W2, b2.reshape(1, H),
                     W3.reshape(1, H), b3)


# speedup vs baseline: 1.1950x; 1.1950x over previous
import functools

import jax
import jax.numpy as jnp
from jax import lax
from jax.experimental import pallas as pl
from jax.experimental.pallas import tpu as pltpu
from jax.experimental.pallas import tpu_sc as plsc

D = 64
H = 256
B = 16384
T = 6
NR = B * T

CIN = 4096
COUT = CIN // 2
GRID_A = 245
VP = GRID_A * COUT


def _pack_body(xt_ref, o_ref):
    x = xt_ref[...]
    row = lax.broadcasted_iota(jnp.int32, (D, D), 0)
    col = lax.broadcasted_iota(jnp.int32, (D, D), 1)
    eye = (row == col).astype(jnp.float32)
    z = lax.dot_general(x, eye, (((0,), (0,)), ((), ())),
                        preferred_element_type=jnp.float32)
    z3 = z.reshape(COUT, 2, D)
    o_ref[:, 0:D] = z3[:, 0, :]
    o_ref[:, D:2 * D] = z3[:, 1, :]


_pack_call = pl.pallas_call(
    _pack_body,
    grid=(GRID_A,),
    in_specs=[pl.BlockSpec((D, CIN), lambda i: (0, i))],
    out_specs=pl.BlockSpec((COUT, 128), lambda i: (i, 0)),
    out_shape=jax.ShapeDtypeStruct((VP, 128), jnp.float32),
)

NC = 2
NS = 16
NW = NC * NS
RW = NR // NW
IDX_MINOR = 128
IDX_SLICES = RW // IDX_MINOR
CHUNK_ROWS = 384
SLICES_PER_CHUNK = CHUNK_ROWS // IDX_MINOR
NCHUNKS = RW // CHUNK_ROWS


def _gather_body(idx_hbm, table_hbm, out_hbm, idx_v, rows_a, rows_b,
                 sem_a, sem_b):
    c = lax.axis_index("c")
    s = lax.axis_index("s")
    wid = s * NC + c
    base_row = wid * RW

    pltpu.sync_copy(idx_hbm.at[wid], idx_v)
    for sl in range(IDX_SLICES):
        for k in range(IDX_MINOR // 16):
            col = pl.ds(k * 16, 16)
            idx_v[sl, col] = lax.shift_right_logical(idx_v[sl, col], 1)

    rows = (rows_a, rows_b)
    sems = (sem_a, sem_b)

    def fire(ci):
        buf = ci % 2
        cps = []
        for j in range(SLICES_PER_CHUNK):
            cps.append(pltpu.async_copy(
                table_hbm.at[idx_v.at[ci * SLICES_PER_CHUNK + j]],
                rows[buf].at[pl.ds(j * IDX_MINOR, IDX_MINOR)],
                sems[buf]))
        return cps

    pending = fire(0)
    for ci in range(NCHUNKS):
        buf = ci % 2
        nxt = fire(ci + 1) if ci + 1 < NCHUNKS else None
        for cp in pending:
            cp.wait()
        pending = nxt
        pltpu.sync_copy(
            rows[buf],
            out_hbm.at[pl.ds(base_row + ci * CHUNK_ROWS, CHUNK_ROWS)])


@functools.cache
def _make_gather_call():
    return pl.kernel(
        _gather_body,
        mesh=plsc.VectorSubcoreMesh(core_axis_name="c", subcore_axis_name="s"),
        out_type=jax.ShapeDtypeStruct((NR, 128), jnp.float32),
        scratch_types=[
            pltpu.VMEM((IDX_SLICES, IDX_MINOR), jnp.int32),
            pltpu.VMEM((CHUNK_ROWS, 128), jnp.float32),
            pltpu.VMEM((CHUNK_ROWS, 128), jnp.float32),
            pltpu.SemaphoreType.DMA,
            pltpu.SemaphoreType.DMA,
        ],
    )


BT = 1024


def _gelu(x):
    return 0.5 * x * (1.0 + lax.erf(x * 0.7071067811865476))


def _mlp_body(x0, x1, x2, x3, x4, x5, idx_ref, w1_ref, b1_ref, w2_ref,
              b2_ref, w3_ref, b3_ref, o_ref):
    idx = idx_ref[...]
    team = jnp.zeros((BT, D), jnp.float32)
    for r, xr in enumerate((x0, x1, x2, x3, x4, x5)):
        x = xr[...]
        half = (idx[:, r:r + 1] & 1).astype(jnp.float32)
        lo = x[:, 0:D]
        hi = x[:, D:2 * D]
        team = team + (lo + (hi - lo) * half)
    team = team * (1.0 / T)
    h = jnp.dot(team, w1_ref[...], preferred_element_type=jnp.float32)
    h = _gelu(h + b1_ref[...])
    h = jnp.dot(h, w2_ref[...], preferred_element_type=jnp.float32)
    h = _gelu(h + b2_ref[...])
    logits = jnp.sum(h * w3_ref[...], axis=1) + b3_ref[0]
    o_ref[...] = 1.0 / (1.0 + jnp.exp(-logits))


GB = B // BT


def _member_spec(r):
    return pl.BlockSpec((BT, 128), lambda i, r=r: (r * GB + i, 0))


_mlp_call = pl.pallas_call(
    _mlp_body,
    grid=(GB,),
    in_specs=[_member_spec(r) for r in range(T)] + [
        pl.BlockSpec((BT, T), lambda i: (i, 0)),
        pl.BlockSpec((D, H), lambda i: (0, 0)),
        pl.BlockSpec((1, H), lambda i: (0, 0)),
        pl.BlockSpec((H, H), lambda i: (0, 0)),
        pl.BlockSpec((1, H), lambda i: (0, 0)),
        pl.BlockSpec((1, H), lambda i: (0, 0)),
        pl.BlockSpec(memory_space=pltpu.SMEM),
    ],
    out_specs=pl.BlockSpec((BT,), lambda i: (i,)),
    out_shape=jax.ShapeDtypeStruct((B,), jnp.float32),
)


def kernel(set_indices, emb_table, W1, b1, W2, b2, W3, b3):
    idx = set_indices.astype(jnp.int32)
    idx_mm = idx.T.reshape(NW, IDX_SLICES, IDX_MINOR)
    packed = _pack_call(emb_table.T)
    rows = _make_gather_call()(idx_mm, packed)
    return _mlp_call(rows, rows, rows, rows, rows, rows, idx,
                     W1, b1.reshape(1, H), W2, b2.reshape(1, H),
                     W3.reshape(1, H), b3)


# speedup vs baseline: 1.9614x; 1.6413x over previous
import functools

import jax
import jax.numpy as jnp
from jax import lax
from jax.experimental import pallas as pl
from jax.experimental.pallas import tpu as pltpu
from jax.experimental.pallas import tpu_sc as plsc

D = 64
H = 256
B = 16384
T = 6
NR = B * T

CP = 4096
GRID_A = 123
VP = GRID_A * CP


def _pack_body(x1_ref, x2_ref, e1_ref, e2_ref, o_ref):
    z1 = lax.dot_general(x1_ref[...], e1_ref[...], (((0,), (0,)), ((), ())),
                         preferred_element_type=jnp.float32)
    z2 = lax.dot_general(x2_ref[...], e2_ref[...], (((0,), (0,)), ((), ())),
                         preferred_element_type=jnp.float32)
    o_ref[...] = z1 + z2


_pack_call = pl.pallas_call(
    _pack_body,
    grid=(GRID_A,),
    in_specs=[
        pl.BlockSpec((D, CP), lambda i: (0, i)),
        pl.BlockSpec((D, CP), lambda i: (0, jnp.minimum(i + GRID_A, 244))),
        pl.BlockSpec((D, 2 * D), lambda i: (0, 0)),
        pl.BlockSpec((D, 2 * D), lambda i: (0, 0)),
    ],
    out_specs=pl.BlockSpec((CP, 128), lambda i: (i, 0)),
    out_shape=jax.ShapeDtypeStruct((VP, 128), jnp.float32),
)

NC = 2
NS = 16
NW = NC * NS
RW = NR // NW
IDX_MINOR = 128
IDX_SLICES = RW // IDX_MINOR
CHUNK_ROWS = 384
SLICES_PER_CHUNK = CHUNK_ROWS // IDX_MINOR
NCHUNKS = RW // CHUNK_ROWS


def _gather_body(idx_hbm, table_hbm, out_hbm, idx_v, rows_a, rows_b,
                 sem_a, sem_b):
    c = lax.axis_index("c")
    s = lax.axis_index("s")
    wid = s * NC + c
    base_row = wid * RW

    pltpu.sync_copy(idx_hbm.at[wid], idx_v)
    for sl in range(IDX_SLICES):
        for k in range(IDX_MINOR // 16):
            col = pl.ds(k * 16, 16)
            v16 = idx_v[sl, col]
            idx_v[sl, col] = jnp.where(v16 >= VP, v16 - VP, v16)

    rows = (rows_a, rows_b)
    sems = (sem_a, sem_b)

    def fire(ci):
        buf = ci % 2
        cps = []
        for j in range(SLICES_PER_CHUNK):
            cps.append(pltpu.async_copy(
                table_hbm.at[idx_v.at[ci * SLICES_PER_CHUNK + j]],
                rows[buf].at[pl.ds(j * IDX_MINOR, IDX_MINOR)],
                sems[buf]))
        return cps

    pending = fire(0)
    for ci in range(NCHUNKS):
        buf = ci % 2
        nxt = fire(ci + 1) if ci + 1 < NCHUNKS else None
        for cp in pending:
            cp.wait()
        pending = nxt
        pltpu.sync_copy(
            rows[buf],
            out_hbm.at[pl.ds(base_row + ci * CHUNK_ROWS, CHUNK_ROWS)])


@functools.cache
def _make_gather_call():
    return pl.kernel(
        _gather_body,
        mesh=plsc.VectorSubcoreMesh(core_axis_name="c", subcore_axis_name="s"),
        out_type=jax.ShapeDtypeStruct((NR, 128), jnp.float32),
        scratch_types=[
            pltpu.VMEM((IDX_SLICES, IDX_MINOR), jnp.int32),
            pltpu.VMEM((CHUNK_ROWS, 128), jnp.float32),
            pltpu.VMEM((CHUNK_ROWS, 128), jnp.float32),
            pltpu.SemaphoreType.DMA,
            pltpu.SemaphoreType.DMA,
        ],
    )


BT = 1024


def _gelu(x):
    return 0.5 * x * (1.0 + lax.erf(x * 0.7071067811865476))


def _mlp_body(x0, x1, x2, x3, x4, x5, idx_ref, w1_ref, b1_ref, w2_ref,
              b2_ref, w3_ref, b3_ref, o_ref):
    idx = idx_ref[...]
    team = jnp.zeros((BT, D), jnp.float32)
    for r, xr in enumerate((x0, x1, x2, x3, x4, x5)):
        x = xr[...]
        half = (idx[:, r:r + 1] >= VP).astype(jnp.float32)
        lo = x[:, 0:D]
        hi = x[:, D:2 * D]
        team = team + (lo + (hi - lo) * half)
    team = team * (1.0 / T)
    h = jnp.dot(team, w1_ref[...], preferred_element_type=jnp.float32)
    h = _gelu(h + b1_ref[...])
    h = jnp.dot(h, w2_ref[...], preferred_element_type=jnp.float32)
    h = _gelu(h + b2_ref[...])
    logits = jnp.sum(h * w3_ref[...], axis=1) + b3_ref[0]
    o_ref[...] = 1.0 / (1.0 + jnp.exp(-logits))


GB = B // BT


def _member_spec(r):
    return pl.BlockSpec((BT, 128), lambda i, r=r: (r * GB + i, 0))


_mlp_call = pl.pallas_call(
    _mlp_body,
    grid=(GB,),
    in_specs=[_member_spec(r) for r in range(T)] + [
        pl.BlockSpec((BT, T), lambda i: (i, 0)),
        pl.BlockSpec((D, H), lambda i: (0, 0)),
        pl.BlockSpec((1, H), lambda i: (0, 0)),
        pl.BlockSpec((H, H), lambda i: (0, 0)),
        pl.BlockSpec((1, H), lambda i: (0, 0)),
        pl.BlockSpec((1, H), lambda i: (0, 0)),
        pl.BlockSpec(memory_space=pltpu.SMEM),
    ],
    out_specs=pl.BlockSpec((BT,), lambda i: (i,)),
    out_shape=jax.ShapeDtypeStruct((B,), jnp.float32),
)


def kernel(set_indices, emb_table, W1, b1, W2, b2, W3, b3):
    idx = set_indices.astype(jnp.int32)
    idx_mm = idx.T.reshape(NW, IDX_SLICES, IDX_MINOR)
    e1 = jnp.eye(D, 2 * D, dtype=jnp.float32)
    e2 = jnp.eye(D, 2 * D, k=D, dtype=jnp.float32)
    packed = _pack_call(emb_table.T, emb_table.T, e1, e2)
    rows = _make_gather_call()(idx_mm, packed)
    return _mlp_call(rows, rows, rows, rows, rows, rows, idx,
                     W1, b1.reshape(1, H), W2, b2.reshape(1, H),
                     W3.reshape(1, H), b3)


# speedup vs baseline: 2.1152x; 1.0784x over previous
import functools

import jax
import jax.numpy as jnp
from jax import lax
from jax.experimental import pallas as pl
from jax.experimental.pallas import tpu as pltpu
from jax.experimental.pallas import tpu_sc as plsc

D = 64
H = 256
B = 16384
T = 6
NR = B * T

CP = 6144
GRID_A = 82
VP = GRID_A * CP


def _pack_body(x1_ref, x2_ref, e1_ref, e2_ref, o_ref):
    z1 = lax.dot_general(x1_ref[...], e1_ref[...], (((0,), (0,)), ((), ())),
                         preferred_element_type=jnp.float32)
    z2 = lax.dot_general(x2_ref[...], e2_ref[...], (((0,), (0,)), ((), ())),
                         preferred_element_type=jnp.float32)
    o_ref[...] = z1 + z2


_pack_call = pl.pallas_call(
    _pack_body,
    grid=(GRID_A,),
    in_specs=[
        pl.BlockSpec((D, CP), lambda i: (0, i)),
        pl.BlockSpec((D, CP), lambda i: (0, jnp.minimum(i + GRID_A, 162))),
        pl.BlockSpec((D, 2 * D), lambda i: (0, 0)),
        pl.BlockSpec((D, 2 * D), lambda i: (0, 0)),
    ],
    out_specs=pl.BlockSpec((CP, 128), lambda i: (i, 0)),
    out_shape=jax.ShapeDtypeStruct((VP, 128), jnp.float32),
)

NC = 2
NS = 16
NW = NC * NS
RW = NR // NW
IDX_MINOR = 128
IDX_SLICES = RW // IDX_MINOR
CHUNK_ROWS = 384
SLICES_PER_CHUNK = CHUNK_ROWS // IDX_MINOR
NCHUNKS = RW // CHUNK_ROWS


def _gather_body(idx_hbm, table_hbm, out_hbm, idx_v, rows_a, rows_b,
                 sem_a, sem_b):
    c = lax.axis_index("c")
    s = lax.axis_index("s")
    wid = s * NC + c
    base_row = wid * RW

    pltpu.sync_copy(idx_hbm.at[wid], idx_v)
    for sl in range(IDX_SLICES):
        for k in range(IDX_MINOR // 16):
            col = pl.ds(k * 16, 16)
            v16 = idx_v[sl, col]
            idx_v[sl, col] = jnp.where(v16 >= VP, v16 - VP, v16)

    rows = (rows_a, rows_b)
    sems = (sem_a, sem_b)

    def fire(ci):
        buf = ci % 2
        cps = []
        for j in range(SLICES_PER_CHUNK):
            cps.append(pltpu.async_copy(
                table_hbm.at[idx_v.at[ci * SLICES_PER_CHUNK + j]],
                rows[buf].at[pl.ds(j * IDX_MINOR, IDX_MINOR)],
                sems[buf]))
        return cps

    pending = fire(0)
    for ci in range(NCHUNKS):
        buf = ci % 2
        nxt = fire(ci + 1) if ci + 1 < NCHUNKS else None
        for cp in pending:
            cp.wait()
        pending = nxt
        pltpu.sync_copy(
            rows[buf],
            out_hbm.at[pl.ds(base_row + ci * CHUNK_ROWS, CHUNK_ROWS)])


@functools.cache
def _make_gather_call():
    return pl.kernel(
        _gather_body,
        mesh=plsc.VectorSubcoreMesh(core_axis_name="c", subcore_axis_name="s"),
        out_type=jax.ShapeDtypeStruct((NR, 128), jnp.float32),
        scratch_types=[
            pltpu.VMEM((IDX_SLICES, IDX_MINOR), jnp.int32),
            pltpu.VMEM((CHUNK_ROWS, 128), jnp.float32),
            pltpu.VMEM((CHUNK_ROWS, 128), jnp.float32),
            pltpu.SemaphoreType.DMA,
            pltpu.SemaphoreType.DMA,
        ],
    )


BT = 2048


def _gelu(x):
    return 0.5 * x * (1.0 + lax.erf(x * 0.7071067811865476))


def _mlp_body(x0, x1, x2, x3, x4, x5, idx_ref, w1_ref, b1_ref, w2_ref,
              b2_ref, w3_ref, b3_ref, o_ref):
    idx = idx_ref[...]
    team = jnp.zeros((BT, D), jnp.float32)
    for r, xr in enumerate((x0, x1, x2, x3, x4, x5)):
        x = xr[...]
        half = (idx[:, r:r + 1] >= VP).astype(jnp.float32)
        lo = x[:, 0:D]
        hi = x[:, D:2 * D]
        team = team + (lo + (hi - lo) * half)
    team = team * (1.0 / T)
    h = jnp.dot(team, w1_ref[...], preferred_element_type=jnp.float32)
    h = _gelu(h + b1_ref[...])
    h = jnp.dot(h, w2_ref[...], preferred_element_type=jnp.float32)
    h = _gelu(h + b2_ref[...])
    logits = jnp.sum(h * w3_ref[...], axis=1) + b3_ref[0]
    o_ref[...] = 1.0 / (1.0 + jnp.exp(-logits))


GB = B // BT


def _member_spec(r):
    return pl.BlockSpec((BT, 128), lambda i, r=r: (r * GB + i, 0))


_mlp_call = pl.pallas_call(
    _mlp_body,
    grid=(GB,),
    in_specs=[_member_spec(r) for r in range(T)] + [
        pl.BlockSpec((BT, T), lambda i: (i, 0)),
        pl.BlockSpec((D, H), lambda i: (0, 0)),
        pl.BlockSpec((1, H), lambda i: (0, 0)),
        pl.BlockSpec((H, H), lambda i: (0, 0)),
        pl.BlockSpec((1, H), lambda i: (0, 0)),
        pl.BlockSpec((1, H), lambda i: (0, 0)),
        pl.BlockSpec(memory_space=pltpu.SMEM),
    ],
    out_specs=pl.BlockSpec((BT,), lambda i: (i,)),
    out_shape=jax.ShapeDtypeStruct((B,), jnp.float32),
)


def kernel(set_indices, emb_table, W1, b1, W2, b2, W3, b3):
    idx = set_indices.astype(jnp.int32)
    idx_mm = idx.T.reshape(NW, IDX_SLICES, IDX_MINOR)
    e1 = jnp.eye(D, 2 * D, dtype=jnp.float32)
    e2 = jnp.eye(D, 2 * D, k=D, dtype=jnp.float32)
    packed = _pack_call(emb_table.T, emb_table.T, e1, e2)
    rows = _make_gather_call()(idx_mm, packed)
    return _mlp_call(rows, rows, rows, rows, rows, rows, idx,
                     W1, b1.reshape(1, H), W2, b2.reshape(1, H),
                     W3.reshape(1, H), b3)


# speedup vs baseline: 2.3103x; 1.0923x over previous
import functools

import jax
import jax.numpy as jnp
from jax import lax
from jax.experimental import pallas as pl
from jax.experimental.pallas import tpu as pltpu
from jax.experimental.pallas import tpu_sc as plsc

D = 64
H = 256
B = 16384
T = 6
NR = B * T

CP = 12288
GRID_A = 41
VP = GRID_A * CP


def _pack_body(x1_ref, x2_ref, e1_ref, e2_ref, o_ref):
    z1 = lax.dot_general(x1_ref[...], e1_ref[...], (((0,), (0,)), ((), ())),
                         preferred_element_type=jnp.float32)
    z2 = lax.dot_general(x2_ref[...], e2_ref[...], (((0,), (0,)), ((), ())),
                         preferred_element_type=jnp.float32)
    o_ref[...] = z1 + z2


_pack_call = pl.pallas_call(
    _pack_body,
    grid=(GRID_A,),
    in_specs=[
        pl.BlockSpec((D, CP), lambda i: (0, i)),
        pl.BlockSpec((D, CP), lambda i: (0, jnp.minimum(i + GRID_A, 81))),
        pl.BlockSpec((D, 2 * D), lambda i: (0, 0)),
        pl.BlockSpec((D, 2 * D), lambda i: (0, 0)),
    ],
    out_specs=pl.BlockSpec((CP, 128), lambda i: (i, 0)),
    out_shape=jax.ShapeDtypeStruct((VP, 128), jnp.float32),
)

NC = 2
NS = 16
NW = NC * NS
RW = NR // NW
IDX_MINOR = 128
IDX_SLICES = RW // IDX_MINOR
CHUNK_ROWS = 384
SLICES_PER_CHUNK = CHUNK_ROWS // IDX_MINOR
NCHUNKS = RW // CHUNK_ROWS


def _gather_body(idx_hbm, table_hbm, out_hbm, idx_v, rows_a, rows_b,
                 sem_a, sem_b):
    c = lax.axis_index("c")
    s = lax.axis_index("s")
    wid = s * NC + c
    base_row = wid * RW

    pltpu.sync_copy(idx_hbm.at[wid], idx_v)
    for sl in range(IDX_SLICES):
        for k in range(IDX_MINOR // 16):
            col = pl.ds(k * 16, 16)
            v16 = idx_v[sl, col]
            idx_v[sl, col] = jnp.where(v16 >= VP, v16 - VP, v16)

    rows = (rows_a, rows_b)
    sems = (sem_a, sem_b)

    def fire(ci):
        buf = ci % 2
        cps = []
        for j in range(SLICES_PER_CHUNK):
            cps.append(pltpu.async_copy(
                table_hbm.at[idx_v.at[ci * SLICES_PER_CHUNK + j]],
                rows[buf].at[pl.ds(j * IDX_MINOR, IDX_MINOR)],
                sems[buf]))
        return cps

    pending = fire(0)
    for ci in range(NCHUNKS):
        buf = ci % 2
        nxt = fire(ci + 1) if ci + 1 < NCHUNKS else None
        for cp in pending:
            cp.wait()
        pending = nxt
        pltpu.sync_copy(
            rows[buf],
            out_hbm.at[pl.ds(base_row + ci * CHUNK_ROWS, CHUNK_ROWS)])


@functools.cache
def _make_gather_call():
    return pl.kernel(
        _gather_body,
        mesh=plsc.VectorSubcoreMesh(core_axis_name="c", subcore_axis_name="s"),
        out_type=jax.ShapeDtypeStruct((NR, 128), jnp.float32),
        scratch_types=[
            pltpu.VMEM((IDX_SLICES, IDX_MINOR), jnp.int32),
            pltpu.VMEM((CHUNK_ROWS, 128), jnp.float32),
            pltpu.VMEM((CHUNK_ROWS, 128), jnp.float32),
            pltpu.SemaphoreType.DMA,
            pltpu.SemaphoreType.DMA,
        ],
    )


BT = 2048


def _gelu(x):
    return 0.5 * x * (1.0 + lax.erf(x * 0.7071067811865476))


def _mlp_body(x0, x1, x2, x3, x4, x5, idx_ref, w1_ref, b1_ref, w2_ref,
              b2_ref, w3_ref, b3_ref, o_ref):
    idx = idx_ref[...]
    team = jnp.zeros((BT, D), jnp.float32)
    for r, xr in enumerate((x0, x1, x2, x3, x4, x5)):
        x = xr[...]
        half = (idx[:, r:r + 1] >= VP).astype(jnp.float32)
        lo = x[:, 0:D]
        hi = x[:, D:2 * D]
        team = team + (lo + (hi - lo) * half)
    team = team * (1.0 / T)
    h = jnp.dot(team, w1_ref[...], preferred_element_type=jnp.float32)
    h = _gelu(h + b1_ref[...])
    h = jnp.dot(h, w2_ref[...], preferred_element_type=jnp.float32)
    h = _gelu(h + b2_ref[...])
    logits = jnp.sum(h * w3_ref[...], axis=1) + b3_ref[0]
    o_ref[...] = 1.0 / (1.0 + jnp.exp(-logits))


GB = B // BT


def _member_spec(r):
    return pl.BlockSpec((BT, 128), lambda i, r=r: (r * GB + i, 0))


_mlp_call = pl.pallas_call(
    _mlp_body,
    grid=(GB,),
    in_specs=[_member_spec(r) for r in range(T)] + [
        pl.BlockSpec((BT, T), lambda i: (i, 0)),
        pl.BlockSpec((D, H), lambda i: (0, 0)),
        pl.BlockSpec((1, H), lambda i: (0, 0)),
        pl.BlockSpec((H, H), lambda i: (0, 0)),
        pl.BlockSpec((1, H), lambda i: (0, 0)),
        pl.BlockSpec((1, H), lambda i: (0, 0)),
        pl.BlockSpec(memory_space=pltpu.SMEM),
    ],
    out_specs=pl.BlockSpec((BT,), lambda i: (i,)),
    out_shape=jax.ShapeDtypeStruct((B,), jnp.float32),
)


def kernel(set_indices, emb_table, W1, b1, W2, b2, W3, b3):
    idx = set_indices.astype(jnp.int32)
    idx_mm = idx.T.reshape(NW, IDX_SLICES, IDX_MINOR)
    e1 = jnp.eye(D, 2 * D, dtype=jnp.float32)
    e2 = jnp.eye(D, 2 * D, k=D, dtype=jnp.float32)
    packed = _pack_call(emb_table.T, emb_table.T, e1, e2)
    rows = _make_gather_call()(idx_mm, packed)
    return _mlp_call(rows, rows, rows, rows, rows, rows, idx,
                     W1, b1.reshape(1, H), W2, b2.reshape(1, H),
                     W3.reshape(1, H), b3)
